# baseline (device time: 16423 ns/iter reference)
import os

import jax
import jax.numpy as jnp
from jax import lax
from jax.experimental import pallas as pl
from jax.experimental.pallas import tpu as pltpu

B = 8
H = 8
D = 64
BS = 16
NB = 64
NPAGES = 128
NPAGES_LOCAL = 64
HD = H * D
BH = B * H
ROWS = NPAGES_LOCAL * BS
SCALE = D ** -0.5
NEG = -1e30

_NO_COMM = bool(int(os.environ.get("SCB_NO_COMM", "0")))


def _body(k_ref, v_ref, qt_ref, lmt_ref, out_ref,
          kvf, kstage, vstage, rbuf, mdbuf, rrem, mdrem,
          dma_sems, send_sems, recv_sems):
    my_x = lax.axis_index("x")
    my_y = lax.axis_index("y")
    my_z = lax.axis_index("z")
    nbr = (1 - my_x, my_y, my_z)

    ck = pltpu.make_async_copy(k_ref, kstage, dma_sems.at[0])
    cv = pltpu.make_async_copy(v_ref, vstage, dma_sems.at[1])
    ck.start()
    cv.start()

    ecol = (lax.broadcasted_iota(jnp.int32, (B, BH), 1) // H
            == lax.broadcasted_iota(jnp.int32, (B, BH), 0)
            ).astype(jnp.float32)
    hm = (lax.broadcasted_iota(jnp.int32, (HD, BH), 0) // D
          == lax.broadcasted_iota(jnp.int32, (HD, BH), 1) % H
          ).astype(jnp.float32)
    r16 = (lax.broadcasted_iota(jnp.int32, (ROWS, NPAGES_LOCAL), 0) // BS
           == lax.broadcasted_iota(jnp.int32, (ROWS, NPAGES_LOCAL), 1)
           ).astype(jnp.float32)

    qmall = (lax.dot_general(
        qt_ref[:, :], ecol, (((1,), (0,)), ((), ())),
        preferred_element_type=jnp.float32,
    ) * hm).astype(jnp.bfloat16)
    lm8 = lax.dot_general(
        r16, lmt_ref[:, :], (((1,), (0,)), ((), ())),
        preferred_element_type=jnp.float32,
    )
    lm = lax.dot_general(
        lm8, ecol, (((1,), (0,)), ((), ())),
        preferred_element_type=jnp.float32,
    )

    ck.wait()
    cv.wait()
    kvf[:, 0:HD] = (kstage[:, :, :, :].reshape(ROWS, HD)
                    .astype(jnp.bfloat16))
    kvf[:, HD:2 * HD] = (vstage[:, :, :, :].reshape(ROWS, HD)
                         .astype(jnp.bfloat16))

    s = lax.dot_general(
        kvf[:, 0:HD], qmall, (((1,), (0,)), ((), ())),
        preferred_element_type=jnp.float32,
    ) * SCALE + lm
    m = jnp.max(s, axis=0, keepdims=True)
    p_ = jnp.where(lm > 0.5 * NEG, jnp.exp(s - m), 0.0)
    d = jnp.sum(p_, axis=0, keepdims=True)
    r = lax.dot_general(
        p_.astype(jnp.bfloat16), kvf[:, HD:2 * HD],
        (((0,), (0,)), ((), ())),
        preferred_element_type=jnp.float32,
    )
    rbuf[:, :] = r
    mdbuf[0:1, :] = m
    mdbuf[1:2, :] = d

    if not _NO_COMM:
        barrier_sem = pltpu.get_barrier_semaphore()
        pl.semaphore_signal(barrier_sem, inc=1, device_id=nbr,
                            device_id_type=pl.DeviceIdType.MESH)
        pl.semaphore_wait(barrier_sem, 1)

        rr = pltpu.make_async_remote_copy(
            src_ref=rbuf, dst_ref=rrem,
            send_sem=send_sems.at[0], recv_sem=recv_sems.at[0],
            device_id=nbr, device_id_type=pl.DeviceIdType.MESH,
        )
        rmd = pltpu.make_async_remote_copy(
            src_ref=mdbuf, dst_ref=mdrem,
            send_sem=send_sems.at[1], recv_sem=recv_sems.at[1],
            device_id=nbr, device_id_type=pl.DeviceIdType.MESH,
        )
        rr.start()
        rmd.start()
        rr.wait()
        rmd.wait()

    m1 = mdbuf[0:1, :]
    d1 = mdbuf[1:2, :]
    m2 = mdrem[0:1, :]
    d2 = mdrem[1:2, :]
    mm = jnp.maximum(m1, m2)
    e1 = jnp.exp(m1 - mm)
    e2 = jnp.exp(m2 - mm)
    den = d1 * e1 + d2 * e2
    e1c = jnp.transpose(e1)
    e2c = jnp.transpose(e2)
    denc = jnp.transpose(den)
    merged = (rbuf[:, :] * e1c + rrem[:, :] * e2c) / denc
    hsel = (lax.broadcasted_iota(jnp.int32, (H, HD), 1) // D
            == lax.broadcasted_iota(jnp.int32, (H, HD), 0)
            ).astype(jnp.float32)
    for i in range(B):
        mi = merged[i * H:(i + 1) * H, :]
        out_ref[i:i + 1, :] = jnp.sum(mi * hsel, axis=0, keepdims=True)


def kernel(Q, K, V, bt, lens):
    q2 = Q.reshape(B, HD)

    my_x = lax.axis_index("x")

    jmask = jnp.arange(NB, dtype=jnp.int32)[None, :] < lens[:, None]
    onehot = (bt[:, :, None] ==
              jnp.arange(NPAGES, dtype=jnp.int32)[None, None, :])
    cnt = jnp.sum(jnp.where(jmask[:, :, None], onehot, False)
                  .astype(jnp.float32), axis=1)
    cnt_my = lax.dynamic_slice(cnt, (0, my_x * NPAGES_LOCAL),
                               (B, NPAGES_LOCAL))
    logm = jnp.where(cnt_my > 0, jnp.log(cnt_my), NEG)
    logmt = logm.T
    qt = q2.T

    out2 = pl.pallas_call(
        _body,
        out_shape=jax.ShapeDtypeStruct((B, HD), jnp.float32),
        in_specs=[
            pl.BlockSpec(memory_space=pl.ANY),
            pl.BlockSpec(memory_space=pl.ANY),
            pl.BlockSpec(memory_space=pltpu.VMEM),
            pl.BlockSpec(memory_space=pltpu.VMEM),
        ],
        out_specs=pl.BlockSpec(memory_space=pltpu.VMEM),
        scratch_shapes=[
            pltpu.VMEM((ROWS, 2 * HD), jnp.bfloat16),
            pltpu.VMEM(K.shape, jnp.float32),
            pltpu.VMEM(V.shape, jnp.float32),
            pltpu.VMEM((BH, HD), jnp.float32),
            pltpu.VMEM((2, BH), jnp.float32),
            pltpu.VMEM((BH, HD), jnp.float32),
            pltpu.VMEM((2, BH), jnp.float32),
            pltpu.SemaphoreType.DMA((2,)),
            pltpu.SemaphoreType.DMA((2,)),
            pltpu.SemaphoreType.DMA((2,)),
        ],
        compiler_params=pltpu.CompilerParams(
            collective_id=None if _NO_COMM else 0
        ),
    )(K, V, qt, logmt)
    return out2.reshape(B, 1, H, D)


# device time: 14202 ns/iter; 1.1564x vs baseline; 1.1564x over previous
import os

import jax
import jax.numpy as jnp
from jax import lax
from jax.experimental import pallas as pl
from jax.experimental.pallas import tpu as pltpu

B = 8
H = 8
D = 64
BS = 16
NB = 64
NPAGES = 128
NPAGES_LOCAL = 64
HD = H * D
BH = B * H
ROWS = NPAGES_LOCAL * BS
SCALE = D ** -0.5
NEG = -1e30

_NO_COMM = bool(int(os.environ.get("SCB_NO_COMM", "0")))


def _body(kvf, qt_ref, lmt_ref, out_ref,
          rbuf, mdbuf, rrem, mdrem, send_sems, recv_sems):
    my_x = lax.axis_index("x")
    my_y = lax.axis_index("y")
    my_z = lax.axis_index("z")
    nbr = (1 - my_x, my_y, my_z)

    ecol = (lax.broadcasted_iota(jnp.int32, (B, BH), 1) // H
            == lax.broadcasted_iota(jnp.int32, (B, BH), 0)
            ).astype(jnp.float32)
    hm = (lax.broadcasted_iota(jnp.int32, (HD, BH), 0) // D
          == lax.broadcasted_iota(jnp.int32, (HD, BH), 1) % H
          ).astype(jnp.float32)
    r16 = (lax.broadcasted_iota(jnp.int32, (ROWS, NPAGES_LOCAL), 0) // BS
           == lax.broadcasted_iota(jnp.int32, (ROWS, NPAGES_LOCAL), 1)
           ).astype(jnp.float32)

    qmall = (lax.dot_general(
        qt_ref[:, :], ecol, (((1,), (0,)), ((), ())),
        preferred_element_type=jnp.float32,
    ) * hm).astype(jnp.bfloat16)
    lm8 = lax.dot_general(
        r16, lmt_ref[:, :], (((1,), (0,)), ((), ())),
        preferred_element_type=jnp.float32,
    )
    lm = lax.dot_general(
        lm8, ecol, (((1,), (0,)), ((), ())),
        preferred_element_type=jnp.float32,
    )

    s = lax.dot_general(
        kvf[:, 0:HD], qmall, (((1,), (0,)), ((), ())),
        preferred_element_type=jnp.float32,
    ) * SCALE + lm
    m = jnp.max(s, axis=0, keepdims=True)
    p_ = jnp.where(lm > 0.5 * NEG, jnp.exp(s - m), 0.0)
    d = jnp.sum(p_, axis=0, keepdims=True)
    r = lax.dot_general(
        p_.astype(jnp.bfloat16), kvf[:, HD:2 * HD],
        (((0,), (0,)), ((), ())),
        preferred_element_type=jnp.float32,
    )
    rbuf[:, :] = r
    mdbuf[0:1, :] = m
    mdbuf[1:2, :] = d

    if not _NO_COMM:
        barrier_sem = pltpu.get_barrier_semaphore()
        pl.semaphore_signal(barrier_sem, inc=1, device_id=nbr,
                            device_id_type=pl.DeviceIdType.MESH)
        pl.semaphore_wait(barrier_sem, 1)

        rr = pltpu.make_async_remote_copy(
            src_ref=rbuf, dst_ref=rrem,
            send_sem=send_sems.at[0], recv_sem=recv_sems.at[0],
            device_id=nbr, device_id_type=pl.DeviceIdType.MESH,
        )
        rmd = pltpu.make_async_remote_copy(
            src_ref=mdbuf, dst_ref=mdrem,
            send_sem=send_sems.at[1], recv_sem=recv_sems.at[1],
            device_id=nbr, device_id_type=pl.DeviceIdType.MESH,
        )
        rr.start()
        rmd.start()
        rr.wait()
        rmd.wait()

    m1 = mdbuf[0:1, :]
    d1 = mdbuf[1:2, :]
    m2 = mdrem[0:1, :]
    d2 = mdrem[1:2, :]
    mm = jnp.maximum(m1, m2)
    e1 = jnp.exp(m1 - mm)
    e2 = jnp.exp(m2 - mm)
    den = d1 * e1 + d2 * e2
    e1c = jnp.transpose(e1)
    e2c = jnp.transpose(e2)
    denc = jnp.transpose(den)
    merged = (rbuf[:, :] * e1c + rrem[:, :] * e2c) / denc
    hsel = (lax.broadcasted_iota(jnp.int32, (H, HD), 1) // D
            == lax.broadcasted_iota(jnp.int32, (H, HD), 0)
            ).astype(jnp.float32)
    for i in range(B):
        mi = merged[i * H:(i + 1) * H, :]
        out_ref[i:i + 1, :] = jnp.sum(mi * hsel, axis=0, keepdims=True)


def kernel(Q, K, V, bt, lens):
    kv2 = jnp.concatenate(
        [K.reshape(ROWS, HD), V.reshape(ROWS, HD)], axis=1
    ).astype(jnp.bfloat16)
    q2 = Q.reshape(B, HD)

    my_x = lax.axis_index("x")

    jmask = jnp.arange(NB, dtype=jnp.int32)[None, :] < lens[:, None]
    onehot = (bt[:, :, None] ==
              jnp.arange(NPAGES, dtype=jnp.int32)[None, None, :])
    cnt = jnp.sum(jnp.where(jmask[:, :, None], onehot, False)
                  .astype(jnp.float32), axis=1)
    cnt_my = lax.dynamic_slice(cnt, (0, my_x * NPAGES_LOCAL),
                               (B, NPAGES_LOCAL))
    logm = jnp.where(cnt_my > 0, jnp.log(cnt_my), NEG)
    logmt = logm.T
    qt = q2.T

    out2 = pl.pallas_call(
        _body,
        out_shape=jax.ShapeDtypeStruct((B, HD), jnp.float32),
        in_specs=[
            pl.BlockSpec(memory_space=pltpu.VMEM),
            pl.BlockSpec(memory_space=pltpu.VMEM),
            pl.BlockSpec(memory_space=pltpu.VMEM),
        ],
        out_specs=pl.BlockSpec(memory_space=pltpu.VMEM),
        scratch_shapes=[
            pltpu.VMEM((BH, HD), jnp.float32),
            pltpu.VMEM((2, BH), jnp.float32),
            pltpu.VMEM((BH, HD), jnp.float32),
            pltpu.VMEM((2, BH), jnp.float32),
            pltpu.SemaphoreType.DMA((2,)),
            pltpu.SemaphoreType.DMA((2,)),
        ],
        compiler_params=pltpu.CompilerParams(
            collective_id=None if _NO_COMM else 0
        ),
    )(kv2, qt, logmt)
    return out2.reshape(B, 1, H, D)


# device time: 13357 ns/iter; 1.2295x vs baseline; 1.0633x over previous
import os

import jax
import jax.numpy as jnp
from jax import lax
from jax.experimental import pallas as pl
from jax.experimental.pallas import tpu as pltpu

B = 8
H = 8
D = 64
BS = 16
NB = 64
NPAGES = 128
NPAGES_LOCAL = 64
HD = H * D
BH = B * H
ROWS = NPAGES_LOCAL * BS
SCALE = D ** -0.5
NEG = -1e30

_NO_COMM = bool(int(os.environ.get("SCB_NO_COMM", "0")))


def _body(kv_ref, qt_ref, lmt_ref, out_ref,
          kvf, rbuf, mdbuf, rrem, mdrem, send_sems, recv_sems):
    my_x = lax.axis_index("x")
    my_y = lax.axis_index("y")
    my_z = lax.axis_index("z")
    nbr = (1 - my_x, my_y, my_z)

    kvf[:, :] = kv_ref[:, :].astype(jnp.bfloat16)

    ecol = (lax.broadcasted_iota(jnp.int32, (B, BH), 1) // H
            == lax.broadcasted_iota(jnp.int32, (B, BH), 0)
            ).astype(jnp.float32)
    hm = (lax.broadcasted_iota(jnp.int32, (HD, BH), 0) // D
          == lax.broadcasted_iota(jnp.int32, (HD, BH), 1) % H
          ).astype(jnp.float32)
    r16 = (lax.broadcasted_iota(jnp.int32, (ROWS, NPAGES_LOCAL), 0) // BS
           == lax.broadcasted_iota(jnp.int32, (ROWS, NPAGES_LOCAL), 1)
           ).astype(jnp.float32)

    qmall = (lax.dot_general(
        qt_ref[:, :], ecol, (((1,), (0,)), ((), ())),
        preferred_element_type=jnp.float32,
    ) * hm).astype(jnp.bfloat16)
    lm8 = lax.dot_general(
        r16, lmt_ref[:, :], (((1,), (0,)), ((), ())),
        preferred_element_type=jnp.float32,
    )
    lm = lax.dot_general(
        lm8, ecol, (((1,), (0,)), ((), ())),
        preferred_element_type=jnp.float32,
    )

    s = lax.dot_general(
        kvf[:, 0:HD], qmall, (((1,), (0,)), ((), ())),
        preferred_element_type=jnp.float32,
    ) * SCALE + lm
    m = jnp.max(s, axis=0, keepdims=True)
    p_ = jnp.where(lm > 0.5 * NEG, jnp.exp(s - m), 0.0)
    d = jnp.sum(p_, axis=0, keepdims=True)
    r = lax.dot_general(
        p_.astype(jnp.bfloat16), kvf[:, HD:2 * HD],
        (((0,), (0,)), ((), ())),
        preferred_element_type=jnp.float32,
    )
    rbuf[:, :] = r
    mdbuf[0:1, :] = m
    mdbuf[1:2, :] = d

    if not _NO_COMM:
        barrier_sem = pltpu.get_barrier_semaphore()
        pl.semaphore_signal(barrier_sem, inc=1, device_id=nbr,
                            device_id_type=pl.DeviceIdType.MESH)
        pl.semaphore_wait(barrier_sem, 1)

        rr = pltpu.make_async_remote_copy(
            src_ref=rbuf, dst_ref=rrem,
            send_sem=send_sems.at[0], recv_sem=recv_sems.at[0],
            device_id=nbr, device_id_type=pl.DeviceIdType.MESH,
        )
        rmd = pltpu.make_async_remote_copy(
            src_ref=mdbuf, dst_ref=mdrem,
            send_sem=send_sems.at[1], recv_sem=recv_sems.at[1],
            device_id=nbr, device_id_type=pl.DeviceIdType.MESH,
        )
        rr.start()
        rmd.start()
        rr.wait()
        rmd.wait()

    m1 = mdbuf[0:1, :]
    d1 = mdbuf[1:2, :]
    m2 = mdrem[0:1, :]
    d2 = mdrem[1:2, :]
    mm = jnp.maximum(m1, m2)
    e1 = jnp.exp(m1 - mm)
    e2 = jnp.exp(m2 - mm)
    den = d1 * e1 + d2 * e2
    e1c = jnp.transpose(e1)
    e2c = jnp.transpose(e2)
    denc = jnp.transpose(den)
    merged = (rbuf[:, :] * e1c + rrem[:, :] * e2c) / denc
    hsel = (lax.broadcasted_iota(jnp.int32, (H, HD), 1) // D
            == lax.broadcasted_iota(jnp.int32, (H, HD), 0)
            ).astype(jnp.float32)
    for i in range(B):
        mi = merged[i * H:(i + 1) * H, :]
        out_ref[i:i + 1, :] = jnp.sum(mi * hsel, axis=0, keepdims=True)


def kernel(Q, K, V, bt, lens):
    kv2 = jnp.concatenate(
        [K.reshape(ROWS, HD), V.reshape(ROWS, HD)], axis=1
    )
    q2 = Q.reshape(B, HD)

    my_x = lax.axis_index("x")

    jmask = jnp.arange(NB, dtype=jnp.int32)[None, :] < lens[:, None]
    onehot = (bt[:, :, None] ==
              jnp.arange(NPAGES, dtype=jnp.int32)[None, None, :])
    cnt = jnp.sum(jnp.where(jmask[:, :, None], onehot, False)
                  .astype(jnp.float32), axis=1)
    cnt_my = lax.dynamic_slice(cnt, (0, my_x * NPAGES_LOCAL),
                               (B, NPAGES_LOCAL))
    logm = jnp.where(cnt_my > 0, jnp.log(cnt_my), NEG)
    logmt = logm.T
    qt = q2.T

    out2 = pl.pallas_call(
        _body,
        out_shape=jax.ShapeDtypeStruct((B, HD), jnp.float32),
        in_specs=[
            pl.BlockSpec(memory_space=pltpu.VMEM),
            pl.BlockSpec(memory_space=pltpu.VMEM),
            pl.BlockSpec(memory_space=pltpu.VMEM),
        ],
        out_specs=pl.BlockSpec(memory_space=pltpu.VMEM),
        scratch_shapes=[
            pltpu.VMEM((ROWS, 2 * HD), jnp.bfloat16),
            pltpu.VMEM((BH, HD), jnp.float32),
            pltpu.VMEM((2, BH), jnp.float32),
            pltpu.VMEM((BH, HD), jnp.float32),
            pltpu.VMEM((2, BH), jnp.float32),
            pltpu.SemaphoreType.DMA((2,)),
            pltpu.SemaphoreType.DMA((2,)),
        ],
        compiler_params=pltpu.CompilerParams(
            collective_id=None if _NO_COMM else 0
        ),
    )(kv2, qt, logmt)
    return out2.reshape(B, 1, H, D)
